# baseline (speedup 1.0000x reference)
import functools

import jax
import jax.numpy as jnp
from jax import lax
from jax.experimental import pallas as pl
from jax.experimental.pallas import tpu as pltpu
from jax.experimental.pallas import tpu_sc as plsc

HIDDEN = 768
BATCH = 4
SEQ = 2048
NC = 2
NS = 16
NW = NC * NS
SPAN = SEQ // NW
P = 8
CHUNK = P * BATCH
NCHUNK = SPAN // P
NBUF = 5
LANES = 16
NSLICE = HIDDEN // LANES


def _emb_body(idx_hbm, pos_hbm, tab_hbm, out_hbm, idx_v, rows_v, pos_v,
              gsem0, gsem1, gsem2, gsem3, gsem4,
              osem0, osem1, osem2, osem3, osem4, psem):
    sid = lax.axis_index("s")
    cc = lax.axis_index("c")
    wid = sid * NC + cc
    span0 = wid * SPAN
    pltpu.sync_copy(idx_hbm.at[wid], idx_v)

    gsem = (gsem0, gsem1, gsem2, gsem3, gsem4)
    osem = (osem0, osem1, osem2, osem3, osem4)
    gd = [None] * NBUF
    od = [[] for _ in range(NBUF)]
    pd = [None]

    def start_gather(c):
        b = c % NBUF
        gd[b] = pltpu.async_copy(tab_hbm.at[idx_v.at[c]], rows_v.at[b], gsem[b])

    def start_pos(c):
        pd[0] = pltpu.async_copy(
            pos_hbm.at[pl.ds(span0 + c * P, P)], pos_v, psem)

    start_gather(0)
    start_pos(0)
    start_gather(1)
    start_gather(2)
    start_gather(3)

    for c in range(NCHUNK):
        b = c % NBUF
        gd[b].wait()
        pd[0].wait()
        if c + 4 < NCHUNK:
            b2 = (c + 4) % NBUF
            for dsc in od[b2]:
                dsc.wait()
            start_gather(c + 4)

        def add_pos(p, carry):
            for j in range(NSLICE):
                sl = pl.ds(j * LANES, LANES)
                x = pos_v[p, sl]
                for bt in range(BATCH):
                    plsc.addupdate(rows_v.at[b, bt * P + p, sl], x)
            return carry

        lax.fori_loop(0, P, add_pos, 0)
        if c + 1 < NCHUNK:
            start_pos(c + 1)
        od[b] = [
            pltpu.async_copy(
                rows_v.at[b, pl.ds(bt * P, P)],
                out_hbm.at[pl.ds(bt * SEQ + span0 + c * P, P)], osem[b])
            for bt in range(BATCH)
        ]
    for lst in od:
        for dsc in lst:
            dsc.wait()


@jax.jit
def _emb(idx, token_table, pos_table):
    mesh = plsc.VectorSubcoreMesh(core_axis_name="c", subcore_axis_name="s")
    f = pl.kernel(
        _emb_body,
        mesh=mesh,
        out_type=jax.ShapeDtypeStruct((BATCH * SEQ, HIDDEN), jnp.float32),
        scratch_types=[
            pltpu.VMEM((NCHUNK, CHUNK), jnp.int32),
            pltpu.VMEM((NBUF, CHUNK, HIDDEN), jnp.float32),
            pltpu.VMEM((P, HIDDEN), jnp.float32),
            pltpu.SemaphoreType.DMA,
            pltpu.SemaphoreType.DMA,
            pltpu.SemaphoreType.DMA,
            pltpu.SemaphoreType.DMA,
            pltpu.SemaphoreType.DMA,
            pltpu.SemaphoreType.DMA,
            pltpu.SemaphoreType.DMA,
            pltpu.SemaphoreType.DMA,
            pltpu.SemaphoreType.DMA,
            pltpu.SemaphoreType.DMA,
            pltpu.SemaphoreType.DMA,
        ],
    )
    return f(idx, pos_table, token_table)


def kernel(input_ids, token_table, pos_table):
    idx = (input_ids.astype(jnp.int32)
           .reshape(BATCH, NW, NCHUNK, P)
           .transpose(1, 2, 0, 3)
           .reshape(NW, NCHUNK, CHUNK))
    out = _emb(idx, token_table, pos_table)
    return out.reshape(BATCH, SEQ, HIDDEN)


# speedup vs baseline: 1.0805x; 1.0805x over previous
import functools

import jax
import jax.numpy as jnp
from jax import lax
from jax.experimental import pallas as pl
from jax.experimental.pallas import tpu as pltpu
from jax.experimental.pallas import tpu_sc as plsc

HIDDEN = 768
BATCH = 4
SEQ = 2048
NC = 2
NS = 16
NW = NC * NS
SPAN = SEQ // NW
P = 8
CHUNK = P * BATCH
NCHUNK = SPAN // P
NBUF = 4
LANES = 16
NSLICE = HIDDEN // LANES


def _emb_body(idx_hbm, pos_hbm, tab_hbm, out_hbm, idx_v, rows_v, pos_v,
              gsem0, gsem1, gsem2, gsem3, osem0, osem1, osem2, osem3,
              psem0, psem1):
    sid = lax.axis_index("s")
    cc = lax.axis_index("c")
    wid = sid * NC + cc
    span0 = wid * SPAN
    pltpu.sync_copy(idx_hbm.at[wid], idx_v)

    gsem = (gsem0, gsem1, gsem2, gsem3)
    osem = (osem0, osem1, osem2, osem3)
    psem = (psem0, psem1)
    gd = [None] * NBUF
    od = [[] for _ in range(NBUF)]
    pd = [None, None]

    def start_gather(c):
        b = c % NBUF
        gd[b] = pltpu.async_copy(tab_hbm.at[idx_v.at[c]], rows_v.at[b], gsem[b])

    def start_pos(c):
        k = c & 1
        pd[k] = pltpu.async_copy(
            pos_hbm.at[pl.ds(span0 + c * P, P)], pos_v.at[k], psem[k])

    start_gather(0)
    start_pos(0)
    start_gather(1)
    start_pos(1)
    start_gather(2)

    for c in range(NCHUNK):
        b = c % NBUF
        k = c & 1
        gd[b].wait()
        pd[k].wait()
        if c + 3 < NCHUNK:
            b2 = (c + 3) % NBUF
            for dsc in od[b2]:
                dsc.wait()
            start_gather(c + 3)

        def add_pos(p, carry):
            for j in range(NSLICE):
                sl = pl.ds(j * LANES, LANES)
                x = pos_v[k, p, sl]
                for bt in range(BATCH):
                    plsc.addupdate(rows_v.at[b, bt * P + p, sl], x)
            return carry

        lax.fori_loop(0, P, add_pos, 0)
        if c + 2 < NCHUNK:
            start_pos(c + 2)
        od[b] = [
            pltpu.async_copy(
                rows_v.at[b, pl.ds(bt * P, P)],
                out_hbm.at[pl.ds(bt * SEQ + span0 + c * P, P)], osem[b])
            for bt in range(BATCH)
        ]
    for lst in od:
        for dsc in lst:
            dsc.wait()


@jax.jit
def _emb(idx, token_table, pos_table):
    mesh = plsc.VectorSubcoreMesh(core_axis_name="c", subcore_axis_name="s")
    f = pl.kernel(
        _emb_body,
        mesh=mesh,
        out_type=jax.ShapeDtypeStruct((BATCH * SEQ, HIDDEN), jnp.float32),
        scratch_types=[
            pltpu.VMEM((NCHUNK, CHUNK), jnp.int32),
            pltpu.VMEM((NBUF, CHUNK, HIDDEN), jnp.float32),
            pltpu.VMEM((2, P, HIDDEN), jnp.float32),
            pltpu.SemaphoreType.DMA,
            pltpu.SemaphoreType.DMA,
            pltpu.SemaphoreType.DMA,
            pltpu.SemaphoreType.DMA,
            pltpu.SemaphoreType.DMA,
            pltpu.SemaphoreType.DMA,
            pltpu.SemaphoreType.DMA,
            pltpu.SemaphoreType.DMA,
            pltpu.SemaphoreType.DMA,
            pltpu.SemaphoreType.DMA,
        ],
    )
    return f(idx, pos_table, token_table)


def kernel(input_ids, token_table, pos_table):
    idx = (input_ids.astype(jnp.int32)
           .reshape(BATCH, NW, NCHUNK, P)
           .transpose(1, 2, 0, 3)
           .reshape(NW, NCHUNK, CHUNK))
    out = _emb(idx, token_table, pos_table)
    return out.reshape(BATCH, SEQ, HIDDEN)
